# baseline (speedup 1.0000x reference)
import functools

import jax
import jax.numpy as jnp
from jax import lax
from jax.experimental import pallas as pl
from jax.experimental.pallas import tpu as pltpu
from jax.experimental.pallas import tpu_sc as plsc

B = 16384
T = 100
V = 1024
D = 16

NC = 2
NS = 16
NW = NC * NS
TOT = B * T
R = TOT // NW
C = 3200
NCHUNK = R // C
CB = C // T
L = 16
NBUF = 2
_OFFS = (0, 16, 32, 48, 64, 80, 84)


def _sc_body(leaves_hbm, tables_hbm, out_hbm,
             lv, pat, idx0, idx1, rows0, rows1,
             gsem0, gsem1, ssem0, ssem1):
    c_id = lax.axis_index("c")
    s_id = lax.axis_index("s")
    wid = s_id * NC + c_id
    base = wid * R

    tables_flat = tables_hbm

    idx_bufs = (idx0, idx1)
    row_bufs = (rows0, rows1)
    gsems = (gsem0, gsem1)
    ssems = (ssem0, ssem1)

    for off in _OFFS:
        pat[pl.ds(off, L)] = (off + lax.iota(jnp.int32, L)) * V

    def stage(cix, idx):
        brow0 = pl.multiple_of((base + cix * C) // T, 8)
        pltpu.sync_copy(leaves_hbm.at[pl.ds(brow0, CB), :], lv)

        def rowb(r, _):
            rt = r * T
            for off in _OFFS:
                sl = pl.ds(off, L)
                idx[pl.ds(rt + off, L)] = lv[r, sl] + pat[sl]
            return 0

        lax.fori_loop(0, CB, rowb, 0)

    def fire_gather(b, idx, rows):
        pltpu.async_copy(tables_flat.at[idx], rows, gsems[b])

    def wait_gather(b, idx, rows):
        pltpu.make_async_copy(tables_flat.at[idx], rows, gsems[b]).wait()

    def fire_store(b, cix, rows):
        row0 = pl.multiple_of(base + cix * C, 8)
        pltpu.async_copy(rows, out_hbm.at[pl.ds(row0, C)], ssems[b])

    def wait_store(b, rows):
        pltpu.make_async_copy(rows, out_hbm.at[pl.ds(0, C)], ssems[b]).wait()

    stage(0, idx_bufs[0])
    fire_gather(0, idx_bufs[0], row_bufs[0])
    stage(1, idx_bufs[1])
    fire_gather(1, idx_bufs[1], row_bufs[1])

    def pairbody(go, carry):
        del carry
        for b in range(NBUF):
            e = go * NBUF + b
            wait_gather(b, idx_bufs[b], row_bufs[b])
            fire_store(b, e, row_bufs[b])

            @pl.when(e + NBUF < NCHUNK)
            def _refill():
                stage(e + NBUF, idx_bufs[b])
                wait_store(b, row_bufs[b])
                fire_gather(b, idx_bufs[b], row_bufs[b])

        return 0

    lax.fori_loop(0, NCHUNK // NBUF, pairbody, 0)
    for b in range(NBUF):
        wait_store(b, row_bufs[b])


def _sc_gather(leaves, tables):
    mesh = plsc.VectorSubcoreMesh(core_axis_name="c", subcore_axis_name="s")
    k = functools.partial(
        pl.kernel,
        mesh=mesh,
        out_type=jax.ShapeDtypeStruct((TOT, D), jnp.float32),
        scratch_types=[
            pltpu.VMEM((CB, T), jnp.int32),
            pltpu.VMEM((T,), jnp.int32),
            pltpu.VMEM((C,), jnp.int32),
            pltpu.VMEM((C,), jnp.int32),
            pltpu.VMEM((C, D), jnp.float32),
            pltpu.VMEM((C, D), jnp.float32),
            pltpu.SemaphoreType.DMA,
            pltpu.SemaphoreType.DMA,
            pltpu.SemaphoreType.DMA,
            pltpu.SemaphoreType.DMA,
        ],
        compiler_params=pltpu.CompilerParams(use_tc_tiling_on_sc=False),
    )(_sc_body)
    return k(leaves, tables)


def _sc_flatten_body(tables_hbm, out_hbm, sem):
    c_id = lax.axis_index("c")
    s_id = lax.axis_index("s")
    wid = s_id * NC + c_id

    def fire(i, _):
        t = wid + i * NW

        @pl.when(t < T)
        def _f():
            pltpu.async_copy(
                tables_hbm.at[t], out_hbm.at[pl.ds(t * V, V)], sem)

        return 0

    lax.fori_loop(0, (T + NW - 1) // NW, fire, 0)

    def drain(i, _):
        t = wid + i * NW

        @pl.when(t < T)
        def _d():
            pltpu.make_async_copy(
                tables_hbm.at[0], out_hbm.at[pl.ds(0, V)], sem).wait()

        return 0

    lax.fori_loop(0, (T + NW - 1) // NW, drain, 0)


def _sc_flatten(tables):
    mesh = plsc.VectorSubcoreMesh(core_axis_name="c", subcore_axis_name="s")
    k = functools.partial(
        pl.kernel,
        mesh=mesh,
        out_type=jax.ShapeDtypeStruct((T * V, D), jnp.float32),
        scratch_types=[pltpu.SemaphoreType.DMA],
        compiler_params=pltpu.CompilerParams(use_tc_tiling_on_sc=False),
    )(_sc_flatten_body)
    return k(tables)


def kernel(leaves, tables):
    out = _sc_gather(leaves, _sc_flatten(tables))
    return out.reshape(B, T * D)


# speedup vs baseline: 1.5348x; 1.5348x over previous
import functools

import jax
import jax.numpy as jnp
from jax import lax
from jax.experimental import pallas as pl
from jax.experimental.pallas import tpu as pltpu
from jax.experimental.pallas import tpu_sc as plsc

B = 16384
T = 100
V = 1024
D = 16

NC = 2
NS = 16
NW = NC * NS
TOT = B * T
R = TOT // NW
C = 1600
NCHUNK = R // C
CB = C // T
L = 16
NBUF = 4
_OFFS = (0, 16, 32, 48, 64, 80, 84)


def _sc_body(leaves_hbm, tables_hbm, out_hbm,
             lv, pat, idx0, idx1, idx2, idx3, rows0, rows1, rows2, rows3,
             gsem0, gsem1, gsem2, gsem3, ssem0, ssem1, ssem2, ssem3):
    c_id = lax.axis_index("c")
    s_id = lax.axis_index("s")
    wid = s_id * NC + c_id
    base = wid * R

    tables_flat = tables_hbm

    idx_bufs = (idx0, idx1, idx2, idx3)
    row_bufs = (rows0, rows1, rows2, rows3)
    gsems = (gsem0, gsem1, gsem2, gsem3)
    ssems = (ssem0, ssem1, ssem2, ssem3)

    for off in _OFFS:
        pat[pl.ds(off, L)] = (off + lax.iota(jnp.int32, L)) * V

    def stage(cix, idx):
        brow0 = pl.multiple_of((base + cix * C) // T, 8)
        pltpu.sync_copy(leaves_hbm.at[pl.ds(brow0, CB), :], lv)

        def rowb(r, _):
            rt = r * T
            for off in _OFFS:
                sl = pl.ds(off, L)
                idx[pl.ds(rt + off, L)] = lv[r, sl] + pat[sl]
            return 0

        lax.fori_loop(0, CB, rowb, 0)

    def fire_gather(b, idx, rows):
        pltpu.async_copy(tables_flat.at[idx], rows, gsems[b])

    def wait_gather(b, idx, rows):
        pltpu.make_async_copy(tables_flat.at[idx], rows, gsems[b]).wait()

    def fire_store(b, cix, rows):
        row0 = pl.multiple_of(base + cix * C, 8)
        pltpu.async_copy(rows, out_hbm.at[pl.ds(row0, C)], ssems[b])

    def wait_store(b, rows):
        pltpu.make_async_copy(rows, out_hbm.at[pl.ds(0, C)], ssems[b]).wait()

    for pb in range(NBUF):
        stage(pb, idx_bufs[pb])
        fire_gather(pb, idx_bufs[pb], row_bufs[pb])

    def pairbody(go, carry):
        del carry
        for b in range(NBUF):
            e = go * NBUF + b
            wait_gather(b, idx_bufs[b], row_bufs[b])
            fire_store(b, e, row_bufs[b])

            @pl.when(e + NBUF < NCHUNK)
            def _refill():
                stage(e + NBUF, idx_bufs[b])
                wait_store(b, row_bufs[b])
                fire_gather(b, idx_bufs[b], row_bufs[b])

        return 0

    lax.fori_loop(0, NCHUNK // NBUF, pairbody, 0)
    for b in range(NBUF):
        wait_store(b, row_bufs[b])


def _sc_gather(leaves, tables):
    mesh = plsc.VectorSubcoreMesh(core_axis_name="c", subcore_axis_name="s")
    k = functools.partial(
        pl.kernel,
        mesh=mesh,
        out_type=jax.ShapeDtypeStruct((TOT, D), jnp.float32),
        scratch_types=[
            pltpu.VMEM((CB, T), jnp.int32),
            pltpu.VMEM((T,), jnp.int32),
            pltpu.VMEM((C,), jnp.int32),
            pltpu.VMEM((C,), jnp.int32),
            pltpu.VMEM((C,), jnp.int32),
            pltpu.VMEM((C,), jnp.int32),
            pltpu.VMEM((C, D), jnp.float32),
            pltpu.VMEM((C, D), jnp.float32),
            pltpu.VMEM((C, D), jnp.float32),
            pltpu.VMEM((C, D), jnp.float32),
            pltpu.SemaphoreType.DMA,
            pltpu.SemaphoreType.DMA,
            pltpu.SemaphoreType.DMA,
            pltpu.SemaphoreType.DMA,
            pltpu.SemaphoreType.DMA,
            pltpu.SemaphoreType.DMA,
            pltpu.SemaphoreType.DMA,
            pltpu.SemaphoreType.DMA,
        ],
        compiler_params=pltpu.CompilerParams(use_tc_tiling_on_sc=False),
    )(_sc_body)
    return k(leaves, tables)


def kernel(leaves, tables):
    out = _sc_gather(leaves, tables.reshape(T * V, D))
    return out.reshape(B, T * D)
